# Initial kernel scaffold; baseline (speedup 1.0000x reference)
import jax
import jax.numpy as jnp
from jax.experimental import pallas as pl


def kernel(unknown, known, unknow_feats, known_feats, grouped_xyz, inds, W1, b1, gamma1, beta1, W2, b2, gamma2, beta2):
    raise NotImplementedError("write your pallas kernel here")


# speedup vs baseline: 14.8596x; 14.8596x over previous
import functools
import jax
import jax.numpy as jnp
from jax import lax
from jax.experimental import pallas as pl
from jax.experimental.pallas import tpu as pltpu

_NBLK = 512
_EPS_BN = 1e-3
_BIG = 3.0e38


def _fused_body(u_ref, kt_ref, uf_ref, kf_ref, w1_ref, b1_ref, g1_ref,
                be1_ref, w2_ref, b2_ref, g2_ref, be2_ref, out_ref):
    u = u_ref[0]
    kt = kt_ref[0]
    M = kt.shape[1]
    N = u.shape[0]

    un2 = jnp.sum(u * u, axis=1, keepdims=True)
    kn2 = jnp.sum(kt * kt, axis=0, keepdims=True)
    cross = jax.lax.dot_general(
        u.astype(jnp.bfloat16), kt.astype(jnp.bfloat16),
        (((1,), (0,)), ((), ())),
        preferred_element_type=jnp.float32)
    d2 = jnp.maximum(un2 + kn2 - 2.0 * cross, 0.0)

    iota = lax.broadcasted_iota(jnp.int32, (N, M), 1)

    dists = []
    onehot = jnp.zeros((N, M), jnp.float32)
    recips = []
    d2w = d2
    for _ in range(3):
        mk = jnp.min(d2w, axis=1, keepdims=True)
        ik = jnp.min(jnp.where(d2w == mk, iota, M), axis=1,
                     keepdims=True)
        rk = 1.0 / (mk + 1e-8)
        recips.append(rk)
        dists.append((mk, ik))
        onehot = onehot + jnp.where(iota == ik, rk, 0.0)
        d2w = jnp.where(iota == ik, _BIG, d2w)

    norm = recips[0] + recips[1] + recips[2]
    wmat = onehot / norm

    kf = kf_ref[0]
    interp = jax.lax.dot_general(
        wmat, kf, (((1,), (0,)), ((), ())),
        preferred_element_type=jnp.float32,
        precision=jax.lax.Precision.HIGHEST)

    uf = uf_ref[0]
    C2 = interp.shape[1]
    w1a = w1_ref[:C2, :]
    w1b = w1_ref[C2:, :]
    x = (jax.lax.dot_general(interp, w1a, (((1,), (0,)), ((), ())),
                             preferred_element_type=jnp.float32,
                             precision=jax.lax.Precision.HIGHEST)
         + jax.lax.dot_general(uf, w1b, (((1,), (0,)), ((), ())),
                               preferred_element_type=jnp.float32,
                               precision=jax.lax.Precision.HIGHEST)
         + b1_ref[0][None, :])
    x = x / jnp.sqrt(1.0 + _EPS_BN) * g1_ref[0][None, :] + be1_ref[0][None, :]
    x = jnp.maximum(x, 0.0)
    x = (jax.lax.dot_general(x, w2_ref[...], (((1,), (0,)), ((), ())),
                             preferred_element_type=jnp.float32,
                             precision=jax.lax.Precision.HIGHEST)
         + b2_ref[0][None, :])
    x = x / jnp.sqrt(1.0 + _EPS_BN) * g2_ref[0][None, :] + be2_ref[0][None, :]
    out_ref[0] = jnp.maximum(x, 0.0)


@jax.jit
def kernel(unknown, known, unknow_feats, known_feats, grouped_xyz, inds,
           W1, b1, gamma1, beta1, W2, b2, gamma2, beta2):
    del grouped_xyz, inds
    B, n, _ = unknown.shape
    m = known.shape[1]
    C1 = unknow_feats.shape[2]
    C2 = known_feats.shape[2]
    H1 = W1.shape[1]
    H2 = W2.shape[1]
    knownT = jnp.swapaxes(known, 1, 2)

    nblk = _NBLK
    grid = (B, n // nblk)

    def row(p):
        return p.reshape(1, -1)

    out = pl.pallas_call(
        _fused_body,
        grid=grid,
        in_specs=[
            pl.BlockSpec((1, nblk, 3), lambda b, j: (b, j, 0)),
            pl.BlockSpec((1, 3, m), lambda b, j: (b, 0, 0)),
            pl.BlockSpec((1, nblk, C1), lambda b, j: (b, j, 0)),
            pl.BlockSpec((1, m, C2), lambda b, j: (b, 0, 0)),
            pl.BlockSpec((C1 + C2, H1), lambda b, j: (0, 0)),
            pl.BlockSpec((1, H1), lambda b, j: (0, 0)),
            pl.BlockSpec((1, H1), lambda b, j: (0, 0)),
            pl.BlockSpec((1, H1), lambda b, j: (0, 0)),
            pl.BlockSpec((H1, H2), lambda b, j: (0, 0)),
            pl.BlockSpec((1, H2), lambda b, j: (0, 0)),
            pl.BlockSpec((1, H2), lambda b, j: (0, 0)),
            pl.BlockSpec((1, H2), lambda b, j: (0, 0)),
        ],
        out_specs=pl.BlockSpec((1, nblk, H2), lambda b, j: (b, j, 0)),
        out_shape=jax.ShapeDtypeStruct((B, n, H2), jnp.float32),
    )(unknown, knownT, unknow_feats, known_feats, W1, row(b1), row(gamma1),
      row(beta1), W2, row(b2), row(gamma2), row(beta2))
    return out


# speedup vs baseline: 31.9899x; 2.1528x over previous
import functools
import jax
import jax.numpy as jnp
from jax import lax
from jax.experimental import pallas as pl
from jax.experimental.pallas import tpu as pltpu

_NBLK = 512
_EPS_BN = 1e-3
_BIG = 3.0e38


def _fused_body(u_ref, kt_ref, uf_ref, kf_ref, w1_ref, b1_ref, g1_ref,
                be1_ref, w2_ref, b2_ref, g2_ref, be2_ref, out_ref):
    u = u_ref[0]
    kt = kt_ref[0]
    M = kt.shape[1]
    N = u.shape[0]

    un2 = jnp.sum(u * u, axis=1, keepdims=True)
    kn2 = jnp.sum(kt * kt, axis=0, keepdims=True)
    cross = jax.lax.dot_general(
        u.astype(jnp.bfloat16), kt.astype(jnp.bfloat16),
        (((1,), (0,)), ((), ())),
        preferred_element_type=jnp.float32)
    d2 = jnp.maximum(un2 + kn2 - 2.0 * cross, 0.0)

    iota = lax.broadcasted_iota(jnp.int32, (N, M), 1)

    dists = []
    onehot = jnp.zeros((N, M), jnp.float32)
    recips = []
    d2w = d2
    for _ in range(3):
        mk = jnp.min(d2w, axis=1, keepdims=True)
        ik = jnp.min(jnp.where(d2w == mk, iota, M), axis=1,
                     keepdims=True)
        rk = 1.0 / (mk + 1e-8)
        recips.append(rk)
        dists.append((mk, ik))
        onehot = onehot + jnp.where(iota == ik, rk, 0.0)
        d2w = jnp.where(iota == ik, _BIG, d2w)

    norm = recips[0] + recips[1] + recips[2]
    wmat = onehot / norm

    kf = kf_ref[0]
    interp = jax.lax.dot_general(
        wmat.astype(jnp.bfloat16), kf.astype(jnp.bfloat16),
        (((1,), (0,)), ((), ())),
        preferred_element_type=jnp.float32)

    uf = uf_ref[0]
    C2 = interp.shape[1]
    w1a = w1_ref[:C2, :]
    w1b = w1_ref[C2:, :]
    x = (jax.lax.dot_general(interp.astype(jnp.bfloat16),
                             w1a.astype(jnp.bfloat16),
                             (((1,), (0,)), ((), ())),
                             preferred_element_type=jnp.float32)
         + jax.lax.dot_general(uf.astype(jnp.bfloat16),
                               w1b.astype(jnp.bfloat16),
                               (((1,), (0,)), ((), ())),
                               preferred_element_type=jnp.float32)
         + b1_ref[0][None, :])
    x = x / jnp.sqrt(1.0 + _EPS_BN) * g1_ref[0][None, :] + be1_ref[0][None, :]
    x = jnp.maximum(x, 0.0)
    x = (jax.lax.dot_general(x.astype(jnp.bfloat16),
                             w2_ref[...].astype(jnp.bfloat16),
                             (((1,), (0,)), ((), ())),
                             preferred_element_type=jnp.float32)
         + b2_ref[0][None, :])
    x = x / jnp.sqrt(1.0 + _EPS_BN) * g2_ref[0][None, :] + be2_ref[0][None, :]
    out_ref[0] = jnp.maximum(x, 0.0)


@jax.jit
def kernel(unknown, known, unknow_feats, known_feats, grouped_xyz, inds,
           W1, b1, gamma1, beta1, W2, b2, gamma2, beta2):
    del grouped_xyz, inds
    B, n, _ = unknown.shape
    m = known.shape[1]
    C1 = unknow_feats.shape[2]
    C2 = known_feats.shape[2]
    H1 = W1.shape[1]
    H2 = W2.shape[1]
    knownT = jnp.swapaxes(known, 1, 2)

    nblk = _NBLK
    grid = (B, n // nblk)

    def row(p):
        return p.reshape(1, -1)

    out = pl.pallas_call(
        _fused_body,
        grid=grid,
        in_specs=[
            pl.BlockSpec((1, nblk, 3), lambda b, j: (b, j, 0)),
            pl.BlockSpec((1, 3, m), lambda b, j: (b, 0, 0)),
            pl.BlockSpec((1, nblk, C1), lambda b, j: (b, j, 0)),
            pl.BlockSpec((1, m, C2), lambda b, j: (b, 0, 0)),
            pl.BlockSpec((C1 + C2, H1), lambda b, j: (0, 0)),
            pl.BlockSpec((1, H1), lambda b, j: (0, 0)),
            pl.BlockSpec((1, H1), lambda b, j: (0, 0)),
            pl.BlockSpec((1, H1), lambda b, j: (0, 0)),
            pl.BlockSpec((H1, H2), lambda b, j: (0, 0)),
            pl.BlockSpec((1, H2), lambda b, j: (0, 0)),
            pl.BlockSpec((1, H2), lambda b, j: (0, 0)),
            pl.BlockSpec((1, H2), lambda b, j: (0, 0)),
        ],
        out_specs=pl.BlockSpec((1, nblk, H2), lambda b, j: (b, j, 0)),
        out_shape=jax.ShapeDtypeStruct((B, n, H2), jnp.float32),
    )(unknown, knownT, unknow_feats, known_feats, W1, row(b1), row(gamma1),
      row(beta1), W2, row(b2), row(gamma2), row(beta2))
    return out


# speedup vs baseline: 45.8125x; 1.4321x over previous
import functools
import jax
import jax.numpy as jnp
from jax import lax
from jax.experimental import pallas as pl
from jax.experimental.pallas import tpu as pltpu

_NBLK = 512
_EPS_BN = 1e-3
_BIG = 3.0e38


def _fused_body(u_ref, kt_ref, uf_ref, kf_ref, w1_ref, b1_ref, g1_ref,
                be1_ref, w2_ref, b2_ref, g2_ref, be2_ref, out_ref):
    u = u_ref[0]
    kt = kt_ref[0]
    M = kt.shape[1]
    N = u.shape[0]

    un2 = jnp.sum(u * u, axis=1, keepdims=True)
    kn2 = jnp.sum(kt * kt, axis=0, keepdims=True)
    cross = jax.lax.dot_general(
        u.astype(jnp.bfloat16), kt.astype(jnp.bfloat16),
        (((1,), (0,)), ((), ())),
        preferred_element_type=jnp.float32)
    d2 = jnp.maximum(un2 + kn2 - 2.0 * cross, 0.0)

    iota_f = lax.broadcasted_iota(jnp.int32, (N, M), 1).astype(jnp.float32)
    d2 = jnp.where(d2 == 0.0, iota_f * 1e-30, d2)

    recips = []
    d2w = d2
    for _ in range(3):
        mk = jnp.min(d2w, axis=1, keepdims=True)
        recips.append(1.0 / (mk + 1e-8))
        d2w = jnp.where(d2w == mk, _BIG, d2w)

    norm = recips[0] + recips[1] + recips[2]
    inv_norm = 1.0 / norm
    wsel = jnp.where(d2w == _BIG, 1.0 / (d2 + 1e-8), 0.0)

    kf = kf_ref[0]
    interp = jax.lax.dot_general(
        wsel.astype(jnp.bfloat16), kf.astype(jnp.bfloat16),
        (((1,), (0,)), ((), ())),
        preferred_element_type=jnp.float32) * inv_norm

    uf = uf_ref[0]
    C2 = interp.shape[1]
    w1a = w1_ref[:C2, :]
    w1b = w1_ref[C2:, :]
    x = (jax.lax.dot_general(interp.astype(jnp.bfloat16),
                             w1a.astype(jnp.bfloat16),
                             (((1,), (0,)), ((), ())),
                             preferred_element_type=jnp.float32)
         + jax.lax.dot_general(uf.astype(jnp.bfloat16),
                               w1b.astype(jnp.bfloat16),
                               (((1,), (0,)), ((), ())),
                               preferred_element_type=jnp.float32)
         + b1_ref[0][None, :])
    x = x / jnp.sqrt(1.0 + _EPS_BN) * g1_ref[0][None, :] + be1_ref[0][None, :]
    x = jnp.maximum(x, 0.0)
    x = (jax.lax.dot_general(x.astype(jnp.bfloat16),
                             w2_ref[...].astype(jnp.bfloat16),
                             (((1,), (0,)), ((), ())),
                             preferred_element_type=jnp.float32)
         + b2_ref[0][None, :])
    x = x / jnp.sqrt(1.0 + _EPS_BN) * g2_ref[0][None, :] + be2_ref[0][None, :]
    out_ref[0] = jnp.maximum(x, 0.0)


@jax.jit
def kernel(unknown, known, unknow_feats, known_feats, grouped_xyz, inds,
           W1, b1, gamma1, beta1, W2, b2, gamma2, beta2):
    del grouped_xyz, inds
    B, n, _ = unknown.shape
    m = known.shape[1]
    C1 = unknow_feats.shape[2]
    C2 = known_feats.shape[2]
    H1 = W1.shape[1]
    H2 = W2.shape[1]
    knownT = jnp.swapaxes(known, 1, 2)

    nblk = _NBLK
    grid = (B, n // nblk)

    def row(p):
        return p.reshape(1, -1)

    out = pl.pallas_call(
        _fused_body,
        grid=grid,
        in_specs=[
            pl.BlockSpec((1, nblk, 3), lambda b, j: (b, j, 0)),
            pl.BlockSpec((1, 3, m), lambda b, j: (b, 0, 0)),
            pl.BlockSpec((1, nblk, C1), lambda b, j: (b, j, 0)),
            pl.BlockSpec((1, m, C2), lambda b, j: (b, 0, 0)),
            pl.BlockSpec((C1 + C2, H1), lambda b, j: (0, 0)),
            pl.BlockSpec((1, H1), lambda b, j: (0, 0)),
            pl.BlockSpec((1, H1), lambda b, j: (0, 0)),
            pl.BlockSpec((1, H1), lambda b, j: (0, 0)),
            pl.BlockSpec((H1, H2), lambda b, j: (0, 0)),
            pl.BlockSpec((1, H2), lambda b, j: (0, 0)),
            pl.BlockSpec((1, H2), lambda b, j: (0, 0)),
            pl.BlockSpec((1, H2), lambda b, j: (0, 0)),
        ],
        out_specs=pl.BlockSpec((1, nblk, H2), lambda b, j: (b, j, 0)),
        out_shape=jax.ShapeDtypeStruct((B, n, H2), jnp.float32),
    )(unknown, knownT, unknow_feats, known_feats, W1, row(b1), row(gamma1),
      row(beta1), W2, row(b2), row(gamma2), row(beta2))
    return out
